# Initial kernel scaffold; baseline (speedup 1.0000x reference)
import jax
import jax.numpy as jnp
from jax import lax
from jax.experimental import pallas as pl
from jax.experimental.pallas import tpu as pltpu
from jax.experimental.pallas import tpu_sc as plsc

NN = 10000
NP = 10240
EE = 320000
DIN = 128
FW = 16
NCLS = 7

NC = 2
NS = 16
NW = NC * NS
CHUNK = 128
EPW = 10112
NCHUNK = EPW // CHUNK
EPAD = NW * EPW
RPW = NP // NS


def _mesh():
    return plsc.VectorSubcoreMesh(
        core_axis_name="c", subcore_axis_name="s", num_cores=NC, num_subcores=NS
    )


def _sc_deg_body(dst_hbm, out_hbm, dst_v, ones_v, stage_v, deg_sp):
    cid = lax.axis_index("c")
    sid = lax.axis_index("s")
    wid = cid * NS + sid

    pltpu.sync_copy(dst_hbm.at[wid], dst_v)

    for k in range(CHUNK // 16):
        ones_v[pl.ds(k * 16, 16)] = jnp.ones((16,), jnp.float32)
    for k in range(RPW // 16):
        stage_v[pl.ds(k * 16, 16)] = jnp.zeros((16,), jnp.float32)

    pltpu.sync_copy(stage_v, deg_sp.at[pl.ds(sid * RPW, RPW)])
    plsc.subcore_barrier()

    def body(j, carry):
        pltpu.sync_copy(ones_v, deg_sp.at[dst_v.at[j]], add=True)
        return carry

    lax.fori_loop(0, NCHUNK, body, 0)
    plsc.subcore_barrier()

    pltpu.sync_copy(deg_sp.at[pl.ds(sid * RPW, RPW)], stage_v)
    pltpu.sync_copy(stage_v, out_hbm.at[cid, pl.ds(sid * RPW, RPW)])


def _sc_deg(dst3):
    return pl.kernel(
        _sc_deg_body,
        out_type=jax.ShapeDtypeStruct((NC, NP), jnp.float32),
        mesh=_mesh(),
        scratch_types=[
            pltpu.VMEM((NCHUNK, CHUNK), jnp.int32),
            pltpu.VMEM((CHUNK,), jnp.float32),
            pltpu.VMEM((RPW,), jnp.float32),
            pltpu.VMEM_SHARED((NP,), jnp.float32),
        ],
    )(dst3)


def _sc_agg_body(src_hbm, dst_hbm, h_hbm, out_hbm, src_v, dst_v, rows_v,
                 stage_v, acc_sp):
    cid = lax.axis_index("c")
    sid = lax.axis_index("s")
    wid = cid * NS + sid

    pltpu.sync_copy(src_hbm.at[wid], src_v)
    pltpu.sync_copy(dst_hbm.at[wid], dst_v)

    for r in range(RPW):
        stage_v[r] = jnp.zeros((FW,), jnp.float32)
    pltpu.sync_copy(stage_v, acc_sp.at[pl.ds(sid * RPW, RPW)])
    plsc.subcore_barrier()

    def body(j, carry):
        pltpu.sync_copy(h_hbm.at[src_v.at[j]], rows_v)
        pltpu.sync_copy(rows_v, acc_sp.at[dst_v.at[j]], add=True)
        return carry

    lax.fori_loop(0, NCHUNK, body, 0)
    plsc.subcore_barrier()

    pltpu.sync_copy(acc_sp.at[pl.ds(sid * RPW, RPW)], stage_v)
    pltpu.sync_copy(stage_v, out_hbm.at[cid, pl.ds(sid * RPW, RPW)])


def _sc_agg(src3, dst3, h):
    return pl.kernel(
        _sc_agg_body,
        out_type=jax.ShapeDtypeStruct((NC, NP, FW), jnp.float32),
        mesh=_mesh(),
        scratch_types=[
            pltpu.VMEM((NCHUNK, CHUNK), jnp.int32),
            pltpu.VMEM((NCHUNK, CHUNK), jnp.int32),
            pltpu.VMEM((CHUNK, FW), jnp.float32),
            pltpu.VMEM((RPW, FW), jnp.float32),
            pltpu.VMEM_SHARED((NP, FW), jnp.float32),
        ],
    )(src3, dst3, h)


def _tc_prep_body(x_ref, w1_ref, degt_ref, h1s_ref, dinv_ref):
    deg = degt_ref[:, 0:1] + degt_ref[:, 1:2] + 1.0
    dinv = lax.rsqrt(deg)
    dinv_ref[...] = dinv
    h = jnp.dot(x_ref[...], w1_ref[...], preferred_element_type=jnp.float32)
    h1s_ref[...] = h * dinv


def _tc_prep(xp, w1, degt):
    return pl.pallas_call(
        _tc_prep_body,
        out_shape=(
            jax.ShapeDtypeStruct((NP, FW), jnp.float32),
            jax.ShapeDtypeStruct((NP, 1), jnp.float32),
        ),
    )(xp, w1, degt)


def _tc_mid_body(p_ref, h1s_ref, dinv_ref, w2_ref, b1_ref, h2s_ref):
    dinv = dinv_ref[...]
    agg = p_ref[0] + p_ref[1] + h1s_ref[...]
    h = jnp.maximum(agg * dinv + b1_ref[...], 0.0)
    h2 = jnp.dot(h, w2_ref[...], preferred_element_type=jnp.float32)
    h2s = h2 * dinv
    rows = lax.broadcasted_iota(jnp.int32, (NP, FW), 0)
    h2s_ref[...] = jnp.where(rows < NN, h2s, 0.0)


def _tc_mid(p, h1s, dinv, w2p, b1r):
    return pl.pallas_call(
        _tc_mid_body,
        out_shape=jax.ShapeDtypeStruct((NP, FW), jnp.float32),
    )(p, h1s, dinv, w2p, b1r)


def _tc_out_body(q_ref, h2s_ref, dinv_ref, b2_ref, out_ref):
    z = (q_ref[0] + q_ref[1] + h2s_ref[...]) * dinv_ref[...]
    z7 = z[:NN, :NCLS] + b2_ref[...]
    m = jnp.max(z7, axis=1, keepdims=True)
    s = z7 - m
    lse = jnp.log(jnp.sum(jnp.exp(s), axis=1, keepdims=True))
    out_ref[...] = s - lse


def _tc_out(q, h2s, dinv, b2r):
    return pl.pallas_call(
        _tc_out_body,
        out_shape=jax.ShapeDtypeStruct((NN, NCLS), jnp.float32),
    )(q, h2s, dinv, b2r)


def kernel(x, edge_index, W1, b1, W2, b2):
    ei = edge_index.astype(jnp.int32)
    pad = NP - NN
    pad_idx = NN + (jnp.arange(EPAD - EE, dtype=jnp.int32) % pad)
    src3 = jnp.concatenate([ei[0], pad_idx]).reshape(NW, NCHUNK, CHUNK)
    dst3 = jnp.concatenate([ei[1], pad_idx]).reshape(NW, NCHUNK, CHUNK)

    xp = jnp.pad(x, ((0, NP - NN), (0, 0)))
    w2p = jnp.pad(W2, ((0, 0), (0, FW - NCLS)))
    b1r = b1.reshape(1, FW)
    b2r = b2.reshape(1, NCLS)

    deg = _sc_deg(dst3)
    degt = deg.T
    h1s, dinv = _tc_prep(xp, W1, degt)
    p = _sc_agg(src3, dst3, h1s)
    h2s = _tc_mid(p, h1s, dinv, w2p, b1r)
    q = _sc_agg(src3, dst3, h2s)
    return _tc_out(q, h2s, dinv, b2r)


# speedup vs baseline: 36.9615x; 36.9615x over previous
import jax
import jax.numpy as jnp
from jax import lax
from jax.experimental import pallas as pl
from jax.experimental.pallas import tpu as pltpu
from jax.experimental.pallas import tpu_sc as plsc

NN = 10000
NP = 10240
EE = 320000
DIN = 128
FW = 16
NCLS = 7

NC = 2
NS = 16
NW = NC * NS
CHUNK = 128
EPW = 10112
NCHUNK = EPW // CHUNK
EPAD = NW * EPW
RPW = NP // NS


def _mesh():
    return plsc.VectorSubcoreMesh(
        core_axis_name="c", subcore_axis_name="s", num_cores=NC, num_subcores=NS
    )


def _sc_deg_body(dst_hbm, out_hbm, dst_v, ones_v, stage_v, deg_sp):
    cid = lax.axis_index("c")
    sid = lax.axis_index("s")
    wid = cid * NS + sid

    pltpu.sync_copy(dst_hbm.at[wid], dst_v)

    for k in range(CHUNK // 16):
        ones_v[pl.ds(k * 16, 16)] = jnp.ones((16,), jnp.float32)
    for k in range(RPW // 16):
        stage_v[pl.ds(k * 16, 16)] = jnp.zeros((16,), jnp.float32)

    pltpu.sync_copy(stage_v, deg_sp.at[pl.ds(sid * RPW, RPW)])
    plsc.subcore_barrier()

    def body(j, carry):
        pltpu.sync_copy(ones_v, deg_sp.at[dst_v.at[j]], add=True)
        return carry

    lax.fori_loop(0, NCHUNK, body, 0)
    plsc.subcore_barrier()

    pltpu.sync_copy(deg_sp.at[pl.ds(sid * RPW, RPW)], stage_v)
    pltpu.sync_copy(stage_v, out_hbm.at[cid, pl.ds(sid * RPW, RPW)])


def _sc_deg(dst3):
    return pl.kernel(
        _sc_deg_body,
        out_type=jax.ShapeDtypeStruct((NC, NP), jnp.float32),
        mesh=_mesh(),
        compiler_params=pltpu.CompilerParams(use_tc_tiling_on_sc=False),
        scratch_types=[
            pltpu.VMEM((NCHUNK, CHUNK), jnp.int32),
            pltpu.VMEM((CHUNK,), jnp.float32),
            pltpu.VMEM((RPW,), jnp.float32),
            pltpu.VMEM_SHARED((NP,), jnp.float32),
        ],
    )(dst3)


def _sc_agg_body(src_hbm, dst_hbm, h_hbm, out_hbm, src_v, dst_v, rows_v,
                 stage_v, acc_sp):
    cid = lax.axis_index("c")
    sid = lax.axis_index("s")
    wid = cid * NS + sid

    pltpu.sync_copy(src_hbm.at[wid], src_v)
    pltpu.sync_copy(dst_hbm.at[wid], dst_v)

    for r in range(64):
        stage_v[r] = jnp.zeros((FW,), jnp.float32)
    for k in range(RPW // 64):
        pltpu.sync_copy(
            stage_v.at[pl.ds(0, 64)], acc_sp.at[pl.ds(sid * RPW + k * 64, 64)]
        )
    plsc.subcore_barrier()

    def body(j, carry):
        pltpu.sync_copy(h_hbm.at[src_v.at[j]], rows_v)
        pltpu.sync_copy(rows_v, acc_sp.at[dst_v.at[j]], add=True)
        return carry

    lax.fori_loop(0, NCHUNK, body, 0)
    plsc.subcore_barrier()

    pltpu.sync_copy(acc_sp.at[pl.ds(sid * RPW, RPW)], stage_v)
    pltpu.sync_copy(stage_v, out_hbm.at[cid, pl.ds(sid * RPW, RPW)])


def _sc_agg(src3, dst3, h):
    return pl.kernel(
        _sc_agg_body,
        out_type=jax.ShapeDtypeStruct((NC, NP, FW), jnp.float32),
        mesh=_mesh(),
        compiler_params=pltpu.CompilerParams(use_tc_tiling_on_sc=False),
        scratch_types=[
            pltpu.VMEM((NCHUNK, CHUNK), jnp.int32),
            pltpu.VMEM((NCHUNK, CHUNK), jnp.int32),
            pltpu.VMEM((CHUNK, FW), jnp.float32),
            pltpu.VMEM((RPW, FW), jnp.float32),
            pltpu.VMEM_SHARED((NP, FW), jnp.float32),
        ],
    )(src3, dst3, h)


def _tc_prep_body(x_ref, w1_ref, degt_ref, h1s_ref, dinv_ref):
    deg = degt_ref[:, 0:1] + degt_ref[:, 1:2] + 1.0
    dinv = lax.rsqrt(deg)
    dinv_ref[...] = dinv
    h = jnp.dot(x_ref[...], w1_ref[...], preferred_element_type=jnp.float32)
    h1s_ref[...] = h * dinv


def _tc_prep(xp, w1, degt):
    return pl.pallas_call(
        _tc_prep_body,
        out_shape=(
            jax.ShapeDtypeStruct((NP, FW), jnp.float32),
            jax.ShapeDtypeStruct((NP, 1), jnp.float32),
        ),
    )(xp, w1, degt)


def _tc_mid_body(p_ref, h1s_ref, dinv_ref, w2_ref, b1_ref, h2s_ref):
    dinv = dinv_ref[...]
    agg = p_ref[0] + p_ref[1] + h1s_ref[...]
    h = jnp.maximum(agg * dinv + b1_ref[...], 0.0)
    h2 = jnp.dot(h, w2_ref[...], preferred_element_type=jnp.float32)
    h2s = h2 * dinv
    rows = lax.broadcasted_iota(jnp.int32, (NP, FW), 0)
    h2s_ref[...] = jnp.where(rows < NN, h2s, 0.0)


def _tc_mid(p, h1s, dinv, w2p, b1r):
    return pl.pallas_call(
        _tc_mid_body,
        out_shape=jax.ShapeDtypeStruct((NP, FW), jnp.float32),
    )(p, h1s, dinv, w2p, b1r)


def _tc_out_body(q_ref, h2s_ref, dinv_ref, b2_ref, out_ref):
    z = (q_ref[0] + q_ref[1] + h2s_ref[...]) * dinv_ref[...]
    z7 = z[:NN, :NCLS] + b2_ref[...]
    m = jnp.max(z7, axis=1, keepdims=True)
    s = z7 - m
    lse = jnp.log(jnp.sum(jnp.exp(s), axis=1, keepdims=True))
    out_ref[...] = s - lse


def _tc_out(q, h2s, dinv, b2r):
    return pl.pallas_call(
        _tc_out_body,
        out_shape=jax.ShapeDtypeStruct((NN, NCLS), jnp.float32),
    )(q, h2s, dinv, b2r)


def kernel(x, edge_index, W1, b1, W2, b2):
    ei = edge_index.astype(jnp.int32)
    pad = NP - NN
    pad_idx = NN + (jnp.arange(EPAD - EE, dtype=jnp.int32) % pad)
    src3 = jnp.concatenate([ei[0], pad_idx]).reshape(NW, NCHUNK, CHUNK)
    dst3 = jnp.concatenate([ei[1], pad_idx]).reshape(NW, NCHUNK, CHUNK)

    xp = jnp.pad(x, ((0, NP - NN), (0, 0)))
    w2p = jnp.pad(W2, ((0, 0), (0, FW - NCLS)))
    b1r = b1.reshape(1, FW)
    b2r = b2.reshape(1, NCLS)

    deg = _sc_deg(dst3)
    degt = deg.T
    h1s, dinv = _tc_prep(xp, W1, degt)
    p = _sc_agg(src3, dst3, h1s)
    h2s = _tc_mid(p, h1s, dinv, w2p, b1r)
    q = _sc_agg(src3, dst3, h2s)
    return _tc_out(q, h2s, dinv, b2r)


# speedup vs baseline: 40.2170x; 1.0881x over previous
import jax
import jax.numpy as jnp
from jax import lax
from jax.experimental import pallas as pl
from jax.experimental.pallas import tpu as pltpu
from jax.experimental.pallas import tpu_sc as plsc

NN = 10000
NP = 10240
EE = 320000
DIN = 128
FW = 16
NCLS = 7

NC = 2
NS = 16
NW = NC * NS
CHUNK = 128
EPW = 10112
NCHUNK = EPW // CHUNK
EPAD = NW * EPW
RPW = NP // NS


def _mesh():
    return plsc.VectorSubcoreMesh(
        core_axis_name="c", subcore_axis_name="s", num_cores=NC, num_subcores=NS
    )


def _sc_deg_body(dst_hbm, out_hbm, dst_v, ones_v, stage_v, deg_sp):
    cid = lax.axis_index("c")
    sid = lax.axis_index("s")
    wid = cid * NS + sid

    pltpu.sync_copy(dst_hbm.at[wid], dst_v)

    for k in range(CHUNK // 16):
        ones_v[pl.ds(k * 16, 16)] = jnp.ones((16,), jnp.float32)
    for k in range(RPW // 16):
        stage_v[pl.ds(k * 16, 16)] = jnp.zeros((16,), jnp.float32)

    pltpu.sync_copy(stage_v, deg_sp.at[pl.ds(sid * RPW, RPW)])
    plsc.subcore_barrier()

    def body(j, carry):
        pltpu.sync_copy(ones_v, deg_sp.at[dst_v.at[j]], add=True)
        return carry

    lax.fori_loop(0, NCHUNK, body, 0)
    plsc.subcore_barrier()

    pltpu.sync_copy(deg_sp.at[pl.ds(sid * RPW, RPW)], stage_v)
    pltpu.sync_copy(stage_v, out_hbm.at[cid, pl.ds(sid * RPW, RPW)])


def _sc_deg(dst3):
    return pl.kernel(
        _sc_deg_body,
        out_type=jax.ShapeDtypeStruct((NC, NP), jnp.float32),
        mesh=_mesh(),
        compiler_params=pltpu.CompilerParams(use_tc_tiling_on_sc=False),
        scratch_types=[
            pltpu.VMEM((NCHUNK, CHUNK), jnp.int32),
            pltpu.VMEM((CHUNK,), jnp.float32),
            pltpu.VMEM((RPW,), jnp.float32),
            pltpu.VMEM_SHARED((NP,), jnp.float32),
        ],
    )(dst3)


def _sc_agg_body(src_hbm, dst_hbm, h_hbm, out_hbm, src_v, dst_v, rows0, rows1,
                 stage_v, acc_sp, sem0, sem1):
    cid = lax.axis_index("c")
    sid = lax.axis_index("s")
    wid = cid * NS + sid

    pltpu.sync_copy(src_hbm.at[wid], src_v)
    pltpu.sync_copy(dst_hbm.at[wid], dst_v)

    for r in range(64):
        stage_v[r] = jnp.zeros((FW,), jnp.float32)
    for k in range(RPW // 64):
        pltpu.sync_copy(
            stage_v.at[pl.ds(0, 64)], acc_sp.at[pl.ds(sid * RPW + k * 64, 64)]
        )
    plsc.subcore_barrier()

    pltpu.async_copy(h_hbm.at[src_v.at[0]], rows0, sem0)

    def pair(i, carry):
        j0 = 2 * i
        j1 = 2 * i + 1
        j2 = jnp.minimum(2 * i + 2, NCHUNK - 1)
        pltpu.make_async_copy(h_hbm.at[src_v.at[j0]], rows0, sem0).wait()
        pltpu.async_copy(h_hbm.at[src_v.at[j1]], rows1, sem1)
        pltpu.sync_copy(rows0, acc_sp.at[dst_v.at[j0]], add=True)
        pltpu.make_async_copy(h_hbm.at[src_v.at[j1]], rows1, sem1).wait()
        pltpu.async_copy(h_hbm.at[src_v.at[j2]], rows0, sem0)
        pltpu.sync_copy(rows1, acc_sp.at[dst_v.at[j1]], add=True)
        return carry

    lax.fori_loop(0, (NCHUNK - 1) // 2, pair, 0)
    pltpu.make_async_copy(h_hbm.at[src_v.at[NCHUNK - 1]], rows0, sem0).wait()
    pltpu.sync_copy(rows0, acc_sp.at[dst_v.at[NCHUNK - 1]], add=True)
    plsc.subcore_barrier()

    pltpu.sync_copy(acc_sp.at[pl.ds(sid * RPW, RPW)], stage_v)
    pltpu.sync_copy(stage_v, out_hbm.at[cid, pl.ds(sid * RPW, RPW)])


def _sc_agg(src3, dst3, h):
    return pl.kernel(
        _sc_agg_body,
        out_type=jax.ShapeDtypeStruct((NC, NP, FW), jnp.float32),
        mesh=_mesh(),
        compiler_params=pltpu.CompilerParams(use_tc_tiling_on_sc=False),
        scratch_types=[
            pltpu.VMEM((NCHUNK, CHUNK), jnp.int32),
            pltpu.VMEM((NCHUNK, CHUNK), jnp.int32),
            pltpu.VMEM((CHUNK, FW), jnp.float32),
            pltpu.VMEM((CHUNK, FW), jnp.float32),
            pltpu.VMEM((RPW, FW), jnp.float32),
            pltpu.VMEM_SHARED((NP, FW), jnp.float32),
            pltpu.SemaphoreType.DMA,
            pltpu.SemaphoreType.DMA,
        ],
    )(src3, dst3, h)


def _tc_prep_body(x_ref, w1_ref, degt_ref, h1s_ref, dinv_ref):
    deg = degt_ref[:, 0:1] + degt_ref[:, 1:2] + 1.0
    dinv = lax.rsqrt(deg)
    dinv_ref[...] = dinv
    h = jnp.dot(x_ref[...], w1_ref[...], preferred_element_type=jnp.float32)
    h1s_ref[...] = h * dinv


def _tc_prep(xp, w1, degt):
    return pl.pallas_call(
        _tc_prep_body,
        out_shape=(
            jax.ShapeDtypeStruct((NP, FW), jnp.float32),
            jax.ShapeDtypeStruct((NP, 1), jnp.float32),
        ),
    )(xp, w1, degt)


def _tc_mid_body(p_ref, h1s_ref, dinv_ref, w2_ref, b1_ref, h2s_ref):
    dinv = dinv_ref[...]
    agg = p_ref[0] + p_ref[1] + h1s_ref[...]
    h = jnp.maximum(agg * dinv + b1_ref[...], 0.0)
    h2 = jnp.dot(h, w2_ref[...], preferred_element_type=jnp.float32)
    h2s = h2 * dinv
    rows = lax.broadcasted_iota(jnp.int32, (NP, FW), 0)
    h2s_ref[...] = jnp.where(rows < NN, h2s, 0.0)


def _tc_mid(p, h1s, dinv, w2p, b1r):
    return pl.pallas_call(
        _tc_mid_body,
        out_shape=jax.ShapeDtypeStruct((NP, FW), jnp.float32),
    )(p, h1s, dinv, w2p, b1r)


def _tc_out_body(q_ref, h2s_ref, dinv_ref, b2_ref, out_ref):
    z = (q_ref[0] + q_ref[1] + h2s_ref[...]) * dinv_ref[...]
    z7 = z[:NN, :NCLS] + b2_ref[...]
    m = jnp.max(z7, axis=1, keepdims=True)
    s = z7 - m
    lse = jnp.log(jnp.sum(jnp.exp(s), axis=1, keepdims=True))
    out_ref[...] = s - lse


def _tc_out(q, h2s, dinv, b2r):
    return pl.pallas_call(
        _tc_out_body,
        out_shape=jax.ShapeDtypeStruct((NN, NCLS), jnp.float32),
    )(q, h2s, dinv, b2r)


def kernel(x, edge_index, W1, b1, W2, b2):
    ei = edge_index.astype(jnp.int32)
    pad = NP - NN
    pad_idx = NN + (jnp.arange(EPAD - EE, dtype=jnp.int32) % pad)
    src3 = jnp.concatenate([ei[0], pad_idx]).reshape(NW, NCHUNK, CHUNK)
    dst3 = jnp.concatenate([ei[1], pad_idx]).reshape(NW, NCHUNK, CHUNK)

    xp = jnp.pad(x, ((0, NP - NN), (0, 0)))
    w2p = jnp.pad(W2, ((0, 0), (0, FW - NCLS)))
    b1r = b1.reshape(1, FW)
    b2r = b2.reshape(1, NCLS)

    deg = _sc_deg(dst3)
    degt = deg.T
    h1s, dinv = _tc_prep(xp, W1, degt)
    p = _sc_agg(src3, dst3, h1s)
    h2s = _tc_mid(p, h1s, dinv, w2p, b1r)
    q = _sc_agg(src3, dst3, h2s)
    return _tc_out(q, h2s, dinv, b2r)


# speedup vs baseline: 49.5453x; 1.2319x over previous
import jax
import jax.numpy as jnp
from jax import lax
from jax.experimental import pallas as pl
from jax.experimental.pallas import tpu as pltpu
from jax.experimental.pallas import tpu_sc as plsc

NN = 10000
NP = 10240
EE = 320000
DIN = 128
FW = 16
NCLS = 7

NC = 2
NS = 16
NW = NC * NS
CHUNK = 128
EPW = 10240
NCHUNK = EPW // CHUNK
EPAD = NW * EPW
RPW = NP // NS


def _mesh():
    return plsc.VectorSubcoreMesh(
        core_axis_name="c", subcore_axis_name="s", num_cores=NC, num_subcores=NS
    )


def _sc_deg_body(dst_hbm, out_hbm, dst_v, ones_v, stage_v, deg_sp, sem0, sem1):
    cid = lax.axis_index("c")
    sid = lax.axis_index("s")
    wid = cid * NS + sid

    pltpu.sync_copy(dst_hbm.at[wid], dst_v)

    for k in range(CHUNK // 16):
        ones_v[pl.ds(k * 16, 16)] = jnp.ones((16,), jnp.float32)
    for k in range(RPW // 16):
        stage_v[pl.ds(k * 16, 16)] = jnp.zeros((16,), jnp.float32)

    pltpu.sync_copy(stage_v, deg_sp.at[pl.ds(sid * RPW, RPW)])
    plsc.subcore_barrier()

    pltpu.async_copy(ones_v, deg_sp.at[dst_v.at[0]], sem0, add=True)
    pltpu.async_copy(ones_v, deg_sp.at[dst_v.at[1]], sem1, add=True)

    def body(i, carry):
        pltpu.make_async_copy(ones_v, deg_sp.at[dst_v.at[0]], sem0).wait()
        pltpu.async_copy(ones_v, deg_sp.at[dst_v.at[2 * i]], sem0, add=True)
        pltpu.make_async_copy(ones_v, deg_sp.at[dst_v.at[0]], sem1).wait()
        pltpu.async_copy(ones_v, deg_sp.at[dst_v.at[2 * i + 1]], sem1, add=True)
        return carry

    lax.fori_loop(1, NCHUNK // 2, body, 0)
    pltpu.make_async_copy(ones_v, deg_sp.at[dst_v.at[0]], sem0).wait()
    pltpu.make_async_copy(ones_v, deg_sp.at[dst_v.at[0]], sem1).wait()
    plsc.subcore_barrier()

    pltpu.sync_copy(deg_sp.at[pl.ds(sid * RPW, RPW)], stage_v)
    pltpu.sync_copy(stage_v, out_hbm.at[cid, pl.ds(sid * RPW, RPW)])


def _sc_deg(dst3):
    return pl.kernel(
        _sc_deg_body,
        out_type=jax.ShapeDtypeStruct((NC, NP), jnp.float32),
        mesh=_mesh(),
        compiler_params=pltpu.CompilerParams(use_tc_tiling_on_sc=False),
        scratch_types=[
            pltpu.VMEM((NCHUNK, CHUNK), jnp.int32),
            pltpu.VMEM((CHUNK,), jnp.float32),
            pltpu.VMEM((RPW,), jnp.float32),
            pltpu.VMEM_SHARED((NP,), jnp.float32),
            pltpu.SemaphoreType.DMA,
            pltpu.SemaphoreType.DMA,
        ],
    )(dst3)


def _sc_agg_body(src_hbm, dst_hbm, h_hbm, out_hbm, src_v, dst_v, rows0, rows1,
                 stage_v, acc_sp, gs0, gs1, ss0, ss1):
    cid = lax.axis_index("c")
    sid = lax.axis_index("s")
    wid = cid * NS + sid

    pltpu.sync_copy(src_hbm.at[wid], src_v)
    pltpu.sync_copy(dst_hbm.at[wid], dst_v)

    for r in range(64):
        stage_v[r] = jnp.zeros((FW,), jnp.float32)
    for k in range(RPW // 64):
        pltpu.sync_copy(
            stage_v.at[pl.ds(0, 64)], acc_sp.at[pl.ds(sid * RPW + k * 64, 64)]
        )
    plsc.subcore_barrier()

    pltpu.async_copy(h_hbm.at[src_v.at[0]], rows0, gs0)
    pltpu.async_copy(h_hbm.at[src_v.at[1]], rows1, gs1)

    def pair(i, carry):
        j0 = 2 * i
        j1 = 2 * i + 1
        pltpu.make_async_copy(h_hbm.at[src_v.at[j0]], rows0, gs0).wait()
        pltpu.async_copy(rows0, acc_sp.at[dst_v.at[j0]], ss0, add=True)
        pltpu.make_async_copy(h_hbm.at[src_v.at[j1]], rows1, gs1).wait()
        pltpu.async_copy(rows1, acc_sp.at[dst_v.at[j1]], ss1, add=True)
        pltpu.make_async_copy(rows0, acc_sp.at[dst_v.at[j0]], ss0).wait()
        pltpu.async_copy(h_hbm.at[src_v.at[j0 + 2]], rows0, gs0)
        pltpu.make_async_copy(rows1, acc_sp.at[dst_v.at[j1]], ss1).wait()
        pltpu.async_copy(h_hbm.at[src_v.at[j1 + 2]], rows1, gs1)
        return carry

    lax.fori_loop(0, NCHUNK // 2 - 1, pair, 0)
    pltpu.make_async_copy(h_hbm.at[src_v.at[NCHUNK - 2]], rows0, gs0).wait()
    pltpu.async_copy(rows0, acc_sp.at[dst_v.at[NCHUNK - 2]], ss0, add=True)
    pltpu.make_async_copy(h_hbm.at[src_v.at[NCHUNK - 1]], rows1, gs1).wait()
    pltpu.async_copy(rows1, acc_sp.at[dst_v.at[NCHUNK - 1]], ss1, add=True)
    pltpu.make_async_copy(rows0, acc_sp.at[dst_v.at[0]], ss0).wait()
    pltpu.make_async_copy(rows1, acc_sp.at[dst_v.at[0]], ss1).wait()
    plsc.subcore_barrier()

    pltpu.sync_copy(acc_sp.at[pl.ds(sid * RPW, RPW)], stage_v)
    pltpu.sync_copy(stage_v, out_hbm.at[cid, pl.ds(sid * RPW, RPW)])


def _sc_agg(src3, dst3, h):
    return pl.kernel(
        _sc_agg_body,
        out_type=jax.ShapeDtypeStruct((NC, NP, FW), jnp.float32),
        mesh=_mesh(),
        compiler_params=pltpu.CompilerParams(use_tc_tiling_on_sc=False),
        scratch_types=[
            pltpu.VMEM((NCHUNK, CHUNK), jnp.int32),
            pltpu.VMEM((NCHUNK, CHUNK), jnp.int32),
            pltpu.VMEM((CHUNK, FW), jnp.float32),
            pltpu.VMEM((CHUNK, FW), jnp.float32),
            pltpu.VMEM((RPW, FW), jnp.float32),
            pltpu.VMEM_SHARED((NP, FW), jnp.float32),
            pltpu.SemaphoreType.DMA,
            pltpu.SemaphoreType.DMA,
            pltpu.SemaphoreType.DMA,
            pltpu.SemaphoreType.DMA,
        ],
    )(src3, dst3, h)


def _tc_prep_body(x_ref, w1_ref, degt_ref, h1s_ref, dinv_ref):
    deg = degt_ref[:, 0:1] + degt_ref[:, 1:2] + 1.0
    dinv = lax.rsqrt(deg)
    dinv_ref[...] = dinv
    h = jnp.dot(x_ref[...], w1_ref[...], preferred_element_type=jnp.float32)
    h1s_ref[...] = h * dinv


def _tc_prep(xp, w1, degt):
    return pl.pallas_call(
        _tc_prep_body,
        out_shape=(
            jax.ShapeDtypeStruct((NP, FW), jnp.float32),
            jax.ShapeDtypeStruct((NP, 1), jnp.float32),
        ),
    )(xp, w1, degt)


def _tc_mid_body(p_ref, h1s_ref, dinv_ref, w2_ref, b1_ref, h2s_ref):
    dinv = dinv_ref[...]
    agg = p_ref[0] + p_ref[1] + h1s_ref[...]
    h = jnp.maximum(agg * dinv + b1_ref[...], 0.0)
    h2 = jnp.dot(h, w2_ref[...], preferred_element_type=jnp.float32)
    h2s = h2 * dinv
    rows = lax.broadcasted_iota(jnp.int32, (NP, FW), 0)
    h2s_ref[...] = jnp.where(rows < NN, h2s, 0.0)


def _tc_mid(p, h1s, dinv, w2p, b1r):
    return pl.pallas_call(
        _tc_mid_body,
        out_shape=jax.ShapeDtypeStruct((NP, FW), jnp.float32),
    )(p, h1s, dinv, w2p, b1r)


def _tc_out_body(q_ref, h2s_ref, dinv_ref, b2_ref, out_ref):
    z = (q_ref[0] + q_ref[1] + h2s_ref[...]) * dinv_ref[...]
    z7 = z[:NN, :NCLS] + b2_ref[...]
    m = jnp.max(z7, axis=1, keepdims=True)
    s = z7 - m
    lse = jnp.log(jnp.sum(jnp.exp(s), axis=1, keepdims=True))
    out_ref[...] = s - lse


def _tc_out(q, h2s, dinv, b2r):
    return pl.pallas_call(
        _tc_out_body,
        out_shape=jax.ShapeDtypeStruct((NN, NCLS), jnp.float32),
    )(q, h2s, dinv, b2r)


def kernel(x, edge_index, W1, b1, W2, b2):
    ei = edge_index.astype(jnp.int32)
    pad = NP - NN
    pad_idx = NN + (jnp.arange(EPAD - EE, dtype=jnp.int32) % pad)
    src3 = jnp.concatenate([ei[0], pad_idx]).reshape(NW, NCHUNK, CHUNK)
    dst3 = jnp.concatenate([ei[1], pad_idx]).reshape(NW, NCHUNK, CHUNK)

    xp = jnp.pad(x, ((0, NP - NN), (0, 0)))
    w2p = jnp.pad(W2, ((0, 0), (0, FW - NCLS)))
    b1r = b1.reshape(1, FW)
    b2r = b2.reshape(1, NCLS)

    deg = _sc_deg(dst3)
    degt = deg.T
    h1s, dinv = _tc_prep(xp, W1, degt)
    p = _sc_agg(src3, dst3, h1s)
    h2s = _tc_mid(p, h1s, dinv, w2p, b1r)
    q = _sc_agg(src3, dst3, h2s)
    return _tc_out(q, h2s, dinv, b2r)
